# baseline (speedup 1.0000x reference)
import jax
import jax.numpy as jnp
from jax.experimental import pallas as pl
from jax.experimental.pallas import tpu as pltpu

_NC = 9
_CS = 1024
_CD = 8
_D = 512
_TB = 1024


def _dot_bf16(a, b):
    return jax.lax.dot_general(
        a.astype(jnp.bfloat16), b.astype(jnp.bfloat16),
        (((1,), (0,)), ((), ())), preferred_element_type=jnp.float32)


def _dot_f32(a, b):
    return jax.lax.dot_general(a, b, (((1,), (0,)), ((), ())),
                               preferred_element_type=jnp.float32)


def _rvq_kernel(z_ref, win_ref, wout_ref, cbn_ref, cbt_ref, ncb_ref,
                inb_ref, outb_ref, zqt_ref, codes_ref, lat_ref, loss_ref,
                res_scr):
    res_scr[...] = z_ref[0]
    zqt_ref[0] = jnp.zeros_like(zqt_ref[0])
    loss = jnp.float32(0.0)
    for i in range(_NC):
        r0 = _CD * i
        wi = win_ref[r0:r0 + _CD, :]
        ze = _dot_bf16(wi, res_scr[...]) + inb_ref[r0:r0 + _CD, :]
        lat_ref[0, r0:r0 + _CD, :] = ze
        nsq = jnp.sum(ze * ze, axis=0, keepdims=True)
        enc_n = ze / jnp.maximum(jnp.sqrt(nsq), 1e-12)
        nself = jnp.sum(enc_n * enc_n, axis=0, keepdims=True)
        cross = _dot_bf16(cbn_ref[i], enc_n)
        dist = (nself - 2.0 * cross) + ncb_ref[_CS * i:_CS * (i + 1), :]
        idx = jnp.argmin(dist, axis=0)
        codes_ref[0, i, :] = idx
        iota = jax.lax.broadcasted_iota(jnp.int32, dist.shape, 0)
        onehot = (iota == idx[None, :]).astype(jnp.float32)
        zq = _dot_f32(cbt_ref[i], onehot)
        d = ze - zq
        loss = loss + jnp.sum(d * d)
        zq_st = ze + (zq - ze)
        wo = wout_ref[:, r0:r0 + _CD]
        out = _dot_bf16(wo, zq_st) + outb_ref[:, i:i + 1]
        zqt_ref[0] = zqt_ref[0] + out
        res_scr[...] = res_scr[...] - out
    loss_ref[0, 0] = jnp.full((8, 128), loss * (1.0 / 1024.0), jnp.float32)


def kernel(z, in_v, in_g, in_b, out_v, out_g, out_b, codebooks):
    B, D, T = z.shape
    nc, cs, cd = codebooks.shape
    f32 = jnp.float32

    n_in = jnp.sqrt(jnp.sum(in_v * in_v, axis=2, keepdims=True))
    win = (in_g[..., None] * in_v / n_in).reshape(nc * cd, D)
    n_out = jnp.sqrt(jnp.sum(out_v * out_v, axis=2, keepdims=True))
    wo3 = out_g[..., None] * out_v / n_out
    wout = jnp.transpose(wo3, (1, 0, 2)).reshape(D, nc * cd)
    cb_norm = jnp.sqrt(jnp.sum(codebooks * codebooks, axis=2, keepdims=True))
    cbn = codebooks / jnp.maximum(cb_norm, 1e-12)
    ncb = jnp.sum(cbn * cbn, axis=2).reshape(nc * cs, 1)
    cbt = jnp.transpose(codebooks, (0, 2, 1))
    inb = in_b.reshape(nc * cd, 1)
    outb = out_b.T

    tb = _TB if T % _TB == 0 else T
    ntb = T // tb
    grid = (B, ntb)
    out_shapes = (
        jax.ShapeDtypeStruct((B, D, T), f32),
        jax.ShapeDtypeStruct((B, nc, T), jnp.int32),
        jax.ShapeDtypeStruct((B, nc * cd, T), f32),
        jax.ShapeDtypeStruct((B, ntb, 8, 128), f32),
    )
    zqt, codes, latents, lpart = pl.pallas_call(
        _rvq_kernel,
        grid=grid,
        in_specs=[
            pl.BlockSpec((1, D, tb), lambda b, t: (b, 0, t)),
            pl.BlockSpec((nc * cd, D), lambda b, t: (0, 0)),
            pl.BlockSpec((D, nc * cd), lambda b, t: (0, 0)),
            pl.BlockSpec((nc, cs, cd), lambda b, t: (0, 0, 0)),
            pl.BlockSpec((nc, cd, cs), lambda b, t: (0, 0, 0)),
            pl.BlockSpec((nc * cs, 1), lambda b, t: (0, 0)),
            pl.BlockSpec((nc * cd, 1), lambda b, t: (0, 0)),
            pl.BlockSpec((D, nc), lambda b, t: (0, 0)),
        ],
        out_specs=[
            pl.BlockSpec((1, D, tb), lambda b, t: (b, 0, t)),
            pl.BlockSpec((1, nc, tb), lambda b, t: (b, 0, t)),
            pl.BlockSpec((1, nc * cd, tb), lambda b, t: (b, 0, t)),
            pl.BlockSpec((1, 1, 8, 128), lambda b, t: (b, t, 0, 0)),
        ],
        out_shape=out_shapes,
        compiler_params=pltpu.CompilerParams(
            dimension_semantics=("parallel", "arbitrary")),
        scratch_shapes=[pltpu.VMEM((D, tb), f32)],
    )(z, win, wout, cbn, cbt, ncb, inb, outb)

    loss = jnp.sum(lpart) * (1.0 / (B * cd * T))
    return (zqt, codes, latents, loss, loss)


# speedup vs baseline: 1.0718x; 1.0718x over previous
import jax
import jax.numpy as jnp
from jax.experimental import pallas as pl
from jax.experimental.pallas import tpu as pltpu

_NC = 9
_CS = 1024
_CD = 8
_D = 512
_TB = 1024


def _dot_bf16(a, b):
    return jax.lax.dot_general(
        a.astype(jnp.bfloat16), b.astype(jnp.bfloat16),
        (((1,), (0,)), ((), ())), preferred_element_type=jnp.float32)


def _dot_f32(a, b):
    return jax.lax.dot_general(a, b, (((1,), (0,)), ((), ())),
                               preferred_element_type=jnp.float32)


def _rvq_kernel(z_ref, win_ref, wout_ref, cbn_ref, cbt_ref, ncb_ref,
                inb_ref, outb_ref, zqt_ref, codes_ref, lat_ref, loss_ref,
                res_scr):
    tb = z_ref.shape[2]
    res_scr[...] = z_ref[0]
    zqt_ref[0] = jnp.zeros_like(zqt_ref[0])
    loss = jnp.float32(0.0)
    ngrp = _CS // 128
    giota = jax.lax.broadcasted_iota(jnp.int32, (ngrp * _CD, tb), 0) // _CD
    for i in range(_NC):
        r0 = _CD * i
        wi = win_ref[r0:r0 + _CD, :]
        ze = _dot_bf16(wi, res_scr[...]) + inb_ref[r0:r0 + _CD, :]
        lat_ref[0, r0:r0 + _CD, :] = ze
        nsq = jnp.sum(ze * ze, axis=0, keepdims=True)
        enc_n = ze / jnp.maximum(jnp.sqrt(nsq), 1e-12)
        nself = jnp.sum(enc_n * enc_n, axis=0, keepdims=True)
        cross = _dot_bf16(cbn_ref[i], enc_n)
        dist = (nself - 2.0 * cross) + ncb_ref[_CS * i:_CS * (i + 1), :]
        idx = jnp.argmin(dist, axis=0)
        codes_ref[0, i, :] = idx
        hi = idx // 128
        lo = jnp.broadcast_to((idx % 128)[None, :], (ngrp * _CD, tb))
        g = jnp.take_along_axis(cbt_ref[i], lo, axis=1)
        gm = jnp.where(jnp.broadcast_to(hi[None, :], giota.shape) == giota,
                       g, 0.0)
        zq = gm[0:_CD, :]
        for k in range(1, ngrp):
            zq = zq + gm[k * _CD:(k + 1) * _CD, :]
        d = ze - zq
        loss = loss + jnp.sum(d * d)
        zq_st = ze + (zq - ze)
        wo = wout_ref[:, r0:r0 + _CD]
        out = _dot_bf16(wo, zq_st) + outb_ref[:, i:i + 1]
        zqt_ref[0] = zqt_ref[0] + out
        res_scr[...] = res_scr[...] - out
    loss_ref[0, 0] = jnp.full((8, 128), loss * (1.0 / 1024.0), jnp.float32)


def kernel(z, in_v, in_g, in_b, out_v, out_g, out_b, codebooks):
    B, D, T = z.shape
    nc, cs, cd = codebooks.shape
    f32 = jnp.float32

    n_in = jnp.sqrt(jnp.sum(in_v * in_v, axis=2, keepdims=True))
    win = (in_g[..., None] * in_v / n_in).reshape(nc * cd, D)
    n_out = jnp.sqrt(jnp.sum(out_v * out_v, axis=2, keepdims=True))
    wo3 = out_g[..., None] * out_v / n_out
    wout = jnp.transpose(wo3, (1, 0, 2)).reshape(D, nc * cd)
    cb_norm = jnp.sqrt(jnp.sum(codebooks * codebooks, axis=2, keepdims=True))
    cbn = codebooks / jnp.maximum(cb_norm, 1e-12)
    ncb = jnp.sum(cbn * cbn, axis=2).reshape(nc * cs, 1)
    ngrp = cs // 128
    cbt = jnp.transpose(codebooks.reshape(nc, ngrp, 128, cd),
                        (0, 1, 3, 2)).reshape(nc, ngrp * cd, 128)
    inb = in_b.reshape(nc * cd, 1)
    outb = out_b.T

    tb = _TB if T % _TB == 0 else T
    ntb = T // tb
    grid = (B, ntb)
    out_shapes = (
        jax.ShapeDtypeStruct((B, D, T), f32),
        jax.ShapeDtypeStruct((B, nc, T), jnp.int32),
        jax.ShapeDtypeStruct((B, nc * cd, T), f32),
        jax.ShapeDtypeStruct((B, ntb, 8, 128), f32),
    )
    zqt, codes, latents, lpart = pl.pallas_call(
        _rvq_kernel,
        grid=grid,
        in_specs=[
            pl.BlockSpec((1, D, tb), lambda b, t: (b, 0, t)),
            pl.BlockSpec((nc * cd, D), lambda b, t: (0, 0)),
            pl.BlockSpec((D, nc * cd), lambda b, t: (0, 0)),
            pl.BlockSpec((nc, cs, cd), lambda b, t: (0, 0, 0)),
            pl.BlockSpec((nc, ngrp * cd, 128), lambda b, t: (0, 0, 0)),
            pl.BlockSpec((nc * cs, 1), lambda b, t: (0, 0)),
            pl.BlockSpec((nc * cd, 1), lambda b, t: (0, 0)),
            pl.BlockSpec((D, nc), lambda b, t: (0, 0)),
        ],
        out_specs=[
            pl.BlockSpec((1, D, tb), lambda b, t: (b, 0, t)),
            pl.BlockSpec((1, nc, tb), lambda b, t: (b, 0, t)),
            pl.BlockSpec((1, nc * cd, tb), lambda b, t: (b, 0, t)),
            pl.BlockSpec((1, 1, 8, 128), lambda b, t: (b, t, 0, 0)),
        ],
        out_shape=out_shapes,
        compiler_params=pltpu.CompilerParams(
            dimension_semantics=("parallel", "arbitrary")),
        scratch_shapes=[pltpu.VMEM((D, tb), f32)],
    )(z, win, wout, cbn, cbt, ncb, inb, outb)

    loss = jnp.sum(lpart) * (1.0 / (B * cd * T))
    return (zqt, codes, latents, loss, loss)


# speedup vs baseline: 1.2807x; 1.1949x over previous
import jax
import jax.numpy as jnp
from jax.experimental import pallas as pl
from jax.experimental.pallas import tpu as pltpu

_NC = 9
_CS = 1024
_CD = 8
_D = 512
_TB = 1024
_K = 16


def _split3(x):
    hi = x.astype(jnp.bfloat16)
    r = x - hi.astype(jnp.float32)
    mid = r.astype(jnp.bfloat16)
    lo = (r - mid.astype(jnp.float32)).astype(jnp.bfloat16)
    return hi, mid, lo


def _rvq_kernel(z_ref, win_ref, wout_ref, cbd_ref, cbt_ref,
                inb_ref, outb_ref, zqt_ref, codes_ref, lat_ref, loss_ref,
                res_scr):
    tb = z_ref.shape[2]
    res_scr[...] = z_ref[0]
    loss = jnp.float32(0.0)
    ngrp = _CS // 128
    giota = jax.lax.broadcasted_iota(jnp.int32, (ngrp * _CD, tb), 0) // _CD
    ones3 = jnp.ones((3, tb), jnp.bfloat16)
    zeros2 = jnp.zeros((2, tb), jnp.bfloat16)
    for i in range(_NC):
        r0 = _CD * i
        wi = win_ref[r0:r0 + _CD, :]
        ze = jax.lax.dot_general(
            wi.astype(jnp.bfloat16), res_scr[...].astype(jnp.bfloat16),
            (((1,), (0,)), ((), ())),
            preferred_element_type=jnp.float32) + inb_ref[r0:r0 + _CD, :]
        lat_ref[0, r0:r0 + _CD, :] = ze
        nsq = jnp.sum(ze * ze, axis=0, keepdims=True)
        enc_n = ze / jnp.maximum(jnp.sqrt(nsq), 1e-12)
        ns = jnp.sum(enc_n * enc_n, axis=0, keepdims=True)
        nhi, nmid, nlo = _split3(ns)
        rhs = jnp.concatenate(
            [enc_n.astype(jnp.bfloat16), ones3, nhi, nmid, nlo, zeros2],
            axis=0)
        dist = jax.lax.dot_general(
            cbd_ref[i], rhs, (((1,), (0,)), ((), ())),
            preferred_element_type=jnp.float32)
        idx = jnp.argmin(dist, axis=0)
        codes_ref[0, i, :] = idx
        hi = idx // 128
        lo = jnp.broadcast_to((idx % 128)[None, :], (ngrp * _CD, tb))
        g = jnp.take_along_axis(cbt_ref[i], lo, axis=1)
        gm = jnp.where(jnp.broadcast_to(hi[None, :], giota.shape) == giota,
                       g, 0.0)
        zq = gm[0:_CD, :]
        for k in range(1, ngrp):
            zq = zq + gm[k * _CD:(k + 1) * _CD, :]
        d = ze - zq
        loss = loss + jnp.sum(d * d)
        zq_st = ze + (zq - ze)
        wo = wout_ref[:, r0:r0 + _CD]
        out = jax.lax.dot_general(
            wo.astype(jnp.bfloat16), zq_st.astype(jnp.bfloat16),
            (((1,), (0,)), ((), ())),
            preferred_element_type=jnp.float32) + outb_ref[:, i:i + 1]
        res_scr[...] = res_scr[...] - out
    zqt_ref[0] = z_ref[0] - res_scr[...]
    loss_ref[0, 0] = jnp.full((8, 128), loss * (1.0 / 1024.0), jnp.float32)


def kernel(z, in_v, in_g, in_b, out_v, out_g, out_b, codebooks):
    B, D, T = z.shape
    nc, cs, cd = codebooks.shape
    f32 = jnp.float32

    n_in = jnp.sqrt(jnp.sum(in_v * in_v, axis=2, keepdims=True))
    win = (in_g[..., None] * in_v / n_in).reshape(nc * cd, D)
    n_out = jnp.sqrt(jnp.sum(out_v * out_v, axis=2, keepdims=True))
    wo3 = out_g[..., None] * out_v / n_out
    wout = jnp.transpose(wo3, (1, 0, 2)).reshape(D, nc * cd)
    cb_norm = jnp.sqrt(jnp.sum(codebooks * codebooks, axis=2, keepdims=True))
    cbn = codebooks / jnp.maximum(cb_norm, 1e-12)
    ncb = jnp.sum(cbn * cbn, axis=2)
    chi, cmid, clo = _split3(ncb)
    cbd = jnp.concatenate(
        [(-2.0 * cbn).astype(jnp.bfloat16),
         chi[..., None], cmid[..., None], clo[..., None],
         jnp.ones((nc, cs, 3), jnp.bfloat16),
         jnp.zeros((nc, cs, 2), jnp.bfloat16)], axis=2)
    ngrp = cs // 128
    cbt = jnp.transpose(codebooks.reshape(nc, ngrp, 128, cd),
                        (0, 1, 3, 2)).reshape(nc, ngrp * cd, 128)
    inb = in_b.reshape(nc * cd, 1)
    outb = out_b.T

    tb = _TB if T % _TB == 0 else T
    ntb = T // tb
    grid = (B, ntb)
    out_shapes = (
        jax.ShapeDtypeStruct((B, D, T), f32),
        jax.ShapeDtypeStruct((B, nc, T), jnp.int32),
        jax.ShapeDtypeStruct((B, nc * cd, T), f32),
        jax.ShapeDtypeStruct((B, ntb, 8, 128), f32),
    )
    zqt, codes, latents, lpart = pl.pallas_call(
        _rvq_kernel,
        grid=grid,
        in_specs=[
            pl.BlockSpec((1, D, tb), lambda b, t: (b, 0, t)),
            pl.BlockSpec((nc * cd, D), lambda b, t: (0, 0)),
            pl.BlockSpec((D, nc * cd), lambda b, t: (0, 0)),
            pl.BlockSpec((nc, cs, _K), lambda b, t: (0, 0, 0)),
            pl.BlockSpec((nc, ngrp * cd, 128), lambda b, t: (0, 0, 0)),
            pl.BlockSpec((nc * cd, 1), lambda b, t: (0, 0)),
            pl.BlockSpec((D, nc), lambda b, t: (0, 0)),
        ],
        out_specs=[
            pl.BlockSpec((1, D, tb), lambda b, t: (b, 0, t)),
            pl.BlockSpec((1, nc, tb), lambda b, t: (b, 0, t)),
            pl.BlockSpec((1, nc * cd, tb), lambda b, t: (b, 0, t)),
            pl.BlockSpec((1, 1, 8, 128), lambda b, t: (b, t, 0, 0)),
        ],
        out_shape=out_shapes,
        compiler_params=pltpu.CompilerParams(
            dimension_semantics=("parallel", "arbitrary")),
        scratch_shapes=[pltpu.VMEM((D, tb), f32)],
    )(z, win, wout, cbd, cbt, inb, outb)

    loss = jnp.sum(lpart) * (1.0 / (B * cd * T))
    return (zqt, codes, latents, loss, loss)
